# baseline (speedup 1.0000x reference)
import functools

import jax
import jax.numpy as jnp
from jax import lax
from jax.experimental import pallas as pl
from jax.experimental.pallas import tpu as pltpu
from jax.experimental.pallas import tpu_sc as plsc

B = 16384
D = 64
N_NEG = 20
L = 16
NC, NS = 2, 16
NW = NC * NS
PER_W = B // NW
C = 32
N_CHUNKS = PER_W // C
NEG_IW = 128
NEG_ROWS_C = C * N_NEG // NEG_IW


def _mf_body(pos_hbm, neg_hbm, upk, itab, out_hbm,
             pidx, nidx, urows, prows, nrows, outv, sems):
    wid = lax.axis_index("s") * NC + lax.axis_index("c")
    base = wid * PER_W

    def fire(c, p):
        off = base + c * C
        pltpu.sync_copy(pos_hbm.at[pl.ds(off, C)], pidx[p])
        for k in range(NEG_ROWS_C):
            pltpu.sync_copy(
                neg_hbm.at[pl.ds(off * N_NEG + k * NEG_IW, NEG_IW)],
                nidx[p].at[k])
        pltpu.async_copy(upk.at[pl.ds(off // 2, C // 2)], urows[p], sems[p])
        pltpu.async_copy(itab.at[pidx[p]], prows[p], sems[p])
        for k in range(NEG_ROWS_C):
            pltpu.async_copy(itab.at[nidx[p].at[k]],
                             nrows[p].at[pl.ds(k * NEG_IW, NEG_IW)],
                             sems[p])

    def wait_all(p):
        pltpu.make_async_copy(upk.at[pl.ds(0, C // 2)], urows[p],
                              sems[p]).wait()
        pltpu.make_async_copy(itab.at[pidx[p]], prows[p], sems[p]).wait()
        for k in range(NEG_ROWS_C):
            pltpu.make_async_copy(itab.at[nidx[p].at[k]],
                                  nrows[p].at[pl.ds(k * NEG_IW, NEG_IW)],
                                  sems[p]).wait()

    def compute(c, p):
        off = base + c * C
        lane = jnp.arange(L, dtype=jnp.int32)
        zero = jnp.zeros((L,), jnp.float32)

        def elem(i, carry):
            ucol = (i & 1) * D
            u = [urows[p][i >> 1, pl.ds(ucol + q * L, L)]
                 for q in range(D // L)]
            pv = [prows[p][i, pl.ds(q * L, L)] for q in range(D // L)]
            pos_sc = jnp.sum(u[0] * pv[0] + u[1] * pv[1]
                             + u[2] * pv[2] + u[3] * pv[3])
            res0 = zero
            res1 = zero
            for j in range(N_NEG):
                r = i * N_NEG + j
                nv = [nrows[p][r, pl.ds(q * L, L)] for q in range(D // L)]
                ns = jnp.sum(u[0] * nv[0] + u[1] * nv[1]
                             + u[2] * nv[2] + u[3] * nv[3])
                r_splat = jnp.full((L,), pos_sc - ns)
                if j < L:
                    res0 = jnp.where(lane == j, r_splat, res0)
                else:
                    res1 = jnp.where(lane == (j - L), r_splat, res1)
            outv[p][i, pl.ds(0, L)] = res0
            plsc.store_scatter(outv[p],
                               [jnp.full((L,), i, jnp.int32),
                                L + (lane & (N_NEG - L - 1))],
                               res1, mask=lane < (N_NEG - L))
            return carry

        lax.fori_loop(0, C, elem, 0)
        pltpu.sync_copy(outv[p], out_hbm.at[pl.ds(off, C)])

    fire(0, 0)

    def pair_body(cp, carry):
        c0 = cp * 2
        fire(c0 + 1, 1)
        wait_all(0)
        compute(c0, 0)

        @pl.when(cp < N_CHUNKS // 2 - 1)
        def _():
            fire(c0 + 2, 0)

        wait_all(1)
        compute(c0 + 1, 1)
        return carry

    lax.fori_loop(0, N_CHUNKS // 2, pair_body, 0)


@jax.jit
def _mf(user, pos_item, neg_flat, user_embed, item_embed):
    mesh = plsc.VectorSubcoreMesh(core_axis_name="c", subcore_axis_name="s",
                                  num_cores=NC, num_subcores=NS)

    user_embed_b, _ = lax.optimization_barrier((user_embed, pos_item))

    u = jnp.take(user_embed_b, user, axis=0)
    u_pk = u.reshape(B // 2, 2 * D)
    run = pl.kernel(
        _mf_body,
        out_type=jax.ShapeDtypeStruct((B, N_NEG), jnp.float32),
        mesh=mesh,
        compiler_params=pltpu.CompilerParams(needs_layout_passes=False,
                                             use_tc_tiling_on_sc=False),
        scratch_types=[
            [pltpu.VMEM((C,), jnp.int32)] * 2,
            [pltpu.VMEM((NEG_ROWS_C, NEG_IW), jnp.int32)] * 2,
            [pltpu.VMEM((C // 2, 2 * D), jnp.float32)] * 2,
            [pltpu.VMEM((C, D), jnp.float32)] * 2,
            [pltpu.VMEM((C * N_NEG, D), jnp.float32)] * 2,
            [pltpu.VMEM((C, N_NEG), jnp.float32)] * 2,
            [pltpu.SemaphoreType.DMA] * 2,
        ],
    )
    return run(pos_item, neg_flat, u_pk, item_embed)


def kernel(user, pos_item, neg_item, user_embed, item_embed):
    user = user.astype(jnp.int32)
    pos_item = pos_item.astype(jnp.int32)
    neg_flat = neg_item.astype(jnp.int32).reshape(B * N_NEG)
    return _mf(user, pos_item, neg_flat, user_embed, item_embed)


# speedup vs baseline: 1.1301x; 1.1301x over previous
import functools

import jax
import jax.numpy as jnp
from jax import lax
from jax.experimental import pallas as pl
from jax.experimental.pallas import tpu as pltpu
from jax.experimental.pallas import tpu_sc as plsc

B = 16384
D = 64
N_NEG = 20
L = 16
NC, NS = 2, 16
NW = NC * NS
PER_W = B // NW
C = 32
N_CHUNKS = PER_W // C
NEG_IW = 128
NEG_ROWS_C = C * N_NEG // NEG_IW


def _mf_body(pos_hbm, neg_hbm, upk, itab, out_hbm,
             pidx, nidx, urows, prows, nrows, outv, sems):
    wid = lax.axis_index("s") * NC + lax.axis_index("c")
    base = wid * PER_W

    def fire(c, p):
        off = base + c * C
        pltpu.sync_copy(pos_hbm.at[pl.ds(off, C)], pidx[p])
        for k in range(NEG_ROWS_C):
            pltpu.sync_copy(
                neg_hbm.at[pl.ds(off * N_NEG + k * NEG_IW, NEG_IW)],
                nidx[p].at[k])
        pltpu.async_copy(upk.at[pl.ds(off // 2, C // 2)], urows[p], sems[p])
        pltpu.async_copy(itab.at[pidx[p]], prows[p], sems[p])
        for k in range(NEG_ROWS_C):
            pltpu.async_copy(itab.at[nidx[p].at[k]],
                             nrows[p].at[pl.ds(k * NEG_IW, NEG_IW)],
                             sems[p])

    def wait_all(p):
        pltpu.make_async_copy(upk.at[pl.ds(0, C // 2)], urows[p],
                              sems[p]).wait()
        pltpu.make_async_copy(itab.at[pidx[p]], prows[p], sems[p]).wait()
        for k in range(NEG_ROWS_C):
            pltpu.make_async_copy(itab.at[nidx[p].at[k]],
                                  nrows[p].at[pl.ds(k * NEG_IW, NEG_IW)],
                                  sems[p]).wait()

    def compute(c, p):
        off = base + c * C
        lane = jnp.arange(L, dtype=jnp.int32)
        zero = jnp.zeros((L,), jnp.float32)

        def elem(i, carry):
            ucol = (i & 1) * D
            u = [urows[p][i >> 1, pl.ds(ucol + q * L, L)]
                 for q in range(D // L)]
            pv = [prows[p][i, pl.ds(q * L, L)] for q in range(D // L)]
            pos_sc = jnp.sum(u[0] * pv[0] + u[1] * pv[1]
                             + u[2] * pv[2] + u[3] * pv[3])
            res0 = zero
            res1 = zero
            for j in range(N_NEG):
                r = i * N_NEG + j
                nv = [nrows[p][r, pl.ds(q * L, L)] for q in range(D // L)]
                ns = jnp.sum(u[0] * nv[0] + u[1] * nv[1]
                             + u[2] * nv[2] + u[3] * nv[3])
                r_splat = jnp.full((L,), pos_sc - ns)
                if j < L:
                    res0 = jnp.where(lane == j, r_splat, res0)
                else:
                    res1 = jnp.where(lane == (j - L), r_splat, res1)
            outv[p][i, pl.ds(0, L)] = res0
            plsc.store_scatter(outv[p],
                               [jnp.full((L,), i, jnp.int32),
                                L + (lane & (N_NEG - L - 1))],
                               res1, mask=lane < (N_NEG - L))
            return carry

        lax.fori_loop(0, C, elem, 0)
        pltpu.sync_copy(outv[p], out_hbm.at[pl.ds(off, C)])

    fire(0, 0)

    def pair_body(cp, carry):
        c0 = cp * 2
        fire(c0 + 1, 1)
        wait_all(0)
        compute(c0, 0)

        @pl.when(cp < N_CHUNKS // 2 - 1)
        def _():
            fire(c0 + 2, 0)

        wait_all(1)
        compute(c0 + 1, 1)
        return carry

    lax.fori_loop(0, N_CHUNKS // 2, pair_body, 0)


@jax.jit
def _mf(user, pos_item, neg_flat, user_embed, item_embed):
    mesh = plsc.VectorSubcoreMesh(core_axis_name="c", subcore_axis_name="s",
                                  num_cores=NC, num_subcores=NS)

    u = jnp.take(user_embed, user, axis=0)
    u_pk = u.reshape(B // 2, 2 * D)
    run = pl.kernel(
        _mf_body,
        out_type=jax.ShapeDtypeStruct((B, N_NEG), jnp.float32),
        mesh=mesh,
        compiler_params=pltpu.CompilerParams(needs_layout_passes=False,
                                             use_tc_tiling_on_sc=False),
        scratch_types=[
            [pltpu.VMEM((C,), jnp.int32)] * 2,
            [pltpu.VMEM((NEG_ROWS_C, NEG_IW), jnp.int32)] * 2,
            [pltpu.VMEM((C // 2, 2 * D), jnp.float32)] * 2,
            [pltpu.VMEM((C, D), jnp.float32)] * 2,
            [pltpu.VMEM((C * N_NEG, D), jnp.float32)] * 2,
            [pltpu.VMEM((C, N_NEG), jnp.float32)] * 2,
            [pltpu.SemaphoreType.DMA] * 2,
        ],
    )
    return run(pos_item, neg_flat, u_pk, item_embed)


def kernel(user, pos_item, neg_item, user_embed, item_embed):
    user = user.astype(jnp.int32)
    pos_item = pos_item.astype(jnp.int32)
    neg_flat = neg_item.astype(jnp.int32).reshape(B * N_NEG)
    return _mf(user, pos_item, neg_flat, user_embed, item_embed)
